# baseline (device time: 252292 ns/iter reference)
import jax
import jax.numpy as jnp
from jax import lax
from jax.experimental import pallas as pl
from jax.experimental.pallas import tpu as pltpu

N_DEV = 32
B, S, C = 4, 512, 256
TAPS = 4
CHUNK = S // N_DEV


def kernel(x, k, Wp):
    def body(x_ref, k_ref, w_ref, out_ref, comm_ref, send_sems, recv_sems,
             credit_sem):
        d = lax.axis_index("i")
        left = jnp.remainder(d - 1, N_DEV)
        right = jnp.remainder(d + 1, N_DEV)

        barrier_sem = pltpu.get_barrier_semaphore()
        for nbr in (left, right):
            pl.semaphore_signal(
                barrier_sem, inc=1,
                device_id=(nbr,), device_id_type=pl.DeviceIdType.MESH,
            )
        pl.semaphore_wait(barrier_sem, 2)

        x_v = x_ref[...]
        xp = jnp.concatenate(
            [jnp.zeros((B, TAPS - 1, C), jnp.float32), x_v], axis=1
        )
        acc = xp[:, 0:S, :] * k_ref[0][None, None, :]
        for t in range(1, TAPS):
            acc = acc + xp[:, t:t + S, :] * k_ref[t][None, None, :]
        a = acc / (1.0 + jnp.exp(-acc))
        partial = jnp.dot(
            a.reshape(B * S, C), w_ref[...],
            preferred_element_type=jnp.float32,
        ).reshape(B, S, C)
        out_ref[...] = partial

        def hop(g, carry):
            slot = lax.rem(g, 2)
            sc = jnp.remainder(d - g, N_DEV)
            rc = jnp.remainder(d - g - 1, N_DEV)

            @pl.when(g >= 2)
            def _():
                pl.semaphore_wait(credit_sem, 1)

            rdma = pltpu.make_async_remote_copy(
                src_ref=out_ref.at[:, pl.ds(sc * CHUNK, CHUNK), :],
                dst_ref=comm_ref.at[slot],
                send_sem=send_sems.at[slot],
                recv_sem=recv_sems.at[slot],
                device_id=(right,),
                device_id_type=pl.DeviceIdType.MESH,
            )
            rdma.start()
            rdma.wait()

            prev = out_ref[:, pl.ds(rc * CHUNK, CHUNK), :]
            contrib = jnp.where(g < N_DEV - 1, prev, jnp.zeros_like(prev))
            out_ref[:, pl.ds(rc * CHUNK, CHUNK), :] = comm_ref[slot] + contrib

            pl.semaphore_signal(
                credit_sem, inc=1,
                device_id=(left,), device_id_type=pl.DeviceIdType.MESH,
            )
            return carry

        lax.fori_loop(0, 2 * (N_DEV - 1), hop, 0)

        pl.semaphore_wait(credit_sem, 2)

    return pl.pallas_call(
        body,
        out_shape=jax.ShapeDtypeStruct((B, S, C), jnp.float32),
        in_specs=[
            pl.BlockSpec(memory_space=pltpu.VMEM),
            pl.BlockSpec(memory_space=pltpu.VMEM),
            pl.BlockSpec(memory_space=pltpu.VMEM),
        ],
        out_specs=pl.BlockSpec(memory_space=pltpu.VMEM),
        scratch_shapes=[
            pltpu.VMEM((2, B, CHUNK, C), jnp.float32),
            pltpu.SemaphoreType.DMA((2,)),
            pltpu.SemaphoreType.DMA((2,)),
            pltpu.SemaphoreType.REGULAR,
        ],
        compiler_params=pltpu.CompilerParams(collective_id=0),
    )(x, k, Wp)


# device time: 72379 ns/iter; 3.4857x vs baseline; 3.4857x over previous
import jax
import jax.numpy as jnp
from jax import lax
from jax.experimental import pallas as pl
from jax.experimental.pallas import tpu as pltpu

N_DEV = 32
B, S, C = 4, 512, 256
TAPS = 4

STAGES = [
    (1, 0),
    (8, 3),
    (2, 1),
    (4, 2),
    (16, 4),
]
HALVES = [S >> (s + 1) for s in range(5)]
COMM_OFF = [sum(HALVES[:s]) for s in range(5)]
COMM_ROWS = sum(HALVES)


def kernel(x, k, Wp):
    def body(x_ref, k_ref, w_ref, out_ref, comm_ref, send_sems, recv_sems):
        d = lax.axis_index("i")

        barrier_sem = pltpu.get_barrier_semaphore()
        for xor_const, _ in STAGES:
            pl.semaphore_signal(
                barrier_sem, inc=1,
                device_id=(d ^ xor_const,),
                device_id_type=pl.DeviceIdType.MESH,
            )
        pl.semaphore_wait(barrier_sem, len(STAGES))

        x_v = x_ref[...]
        xp = jnp.concatenate(
            [jnp.zeros((B, TAPS - 1, C), jnp.float32), x_v], axis=1
        )
        acc = xp[:, 0:S, :] * k_ref[0][None, None, :]
        for t in range(1, TAPS):
            acc = acc + xp[:, t:t + S, :] * k_ref[t][None, None, :]
        a = acc / (1.0 + jnp.exp(-acc))
        partial = jnp.dot(
            a.reshape(B * S, C), w_ref[...],
            preferred_element_type=jnp.float32,
        ).reshape(B, S, C)
        out_ref[...] = partial

        start = jnp.int32(0)
        for s, (xor_const, shift) in enumerate(STAGES):
            h = HALVES[s]
            bit = (d >> shift) & 1
            keep_start = start + bit * h
            send_start = start + (1 - bit) * h
            rdma = pltpu.make_async_remote_copy(
                src_ref=out_ref.at[:, pl.ds(send_start, h), :],
                dst_ref=comm_ref.at[:, pl.ds(COMM_OFF[s], h), :],
                send_sem=send_sems.at[s],
                recv_sem=recv_sems.at[s],
                device_id=(d ^ xor_const,),
                device_id_type=pl.DeviceIdType.MESH,
            )
            rdma.start()
            rdma.wait()
            out_ref[:, pl.ds(keep_start, h), :] = (
                out_ref[:, pl.ds(keep_start, h), :]
                + comm_ref[:, pl.ds(COMM_OFF[s], h), :]
            )
            start = keep_start

        size = S // N_DEV
        for s in range(4, -1, -1):
            xor_const, shift = STAGES[s]
            bit = (d >> shift) & 1
            rdma = pltpu.make_async_remote_copy(
                src_ref=out_ref.at[:, pl.ds(start, size), :],
                dst_ref=out_ref.at[:, pl.ds(start, size), :],
                send_sem=send_sems.at[5 + s],
                recv_sem=recv_sems.at[5 + s],
                device_id=(d ^ xor_const,),
                device_id_type=pl.DeviceIdType.MESH,
            )
            rdma.start()
            rdma.wait()
            start = start - bit * size
            size = 2 * size

    return pl.pallas_call(
        body,
        out_shape=jax.ShapeDtypeStruct((B, S, C), jnp.float32),
        in_specs=[
            pl.BlockSpec(memory_space=pltpu.VMEM),
            pl.BlockSpec(memory_space=pltpu.VMEM),
            pl.BlockSpec(memory_space=pltpu.VMEM),
        ],
        out_specs=pl.BlockSpec(memory_space=pltpu.VMEM),
        scratch_shapes=[
            pltpu.VMEM((B, COMM_ROWS, C), jnp.float32),
            pltpu.SemaphoreType.DMA((10,)),
            pltpu.SemaphoreType.DMA((10,)),
        ],
        compiler_params=pltpu.CompilerParams(collective_id=0),
    )(x, k, Wp)


# device time: 52609 ns/iter; 4.7956x vs baseline; 1.3758x over previous
import jax
import jax.numpy as jnp
from jax import lax
from jax.experimental import pallas as pl
from jax.experimental.pallas import tpu as pltpu

N_DEV = 32
B, S, C = 4, 512, 256
TAPS = 4

N_CHUNK = 2
ROWS = S // N_CHUNK

X, DIAG, Y2, Z, Z2 = (1, 0), (2, 1), (4, 2), (8, 3), (16, 4)
ORDERS = [
    [X, Z, DIAG, Y2, Z2],
    [Z, X, Z2, DIAG, Y2],
]
HALVES = [ROWS >> (s + 1) for s in range(5)]
COMM_OFF = [sum(HALVES[:s]) for s in range(5)]
COMM_ROWS = sum(HALVES)
N_OPS = 10


def kernel(x, k, Wp):
    def body(x_ref, k_ref, w_ref, out_ref, comm_ref, send_sems, recv_sems):
        d = lax.axis_index("i")

        barrier_sem = pltpu.get_barrier_semaphore()
        for xor_const, _ in [X, DIAG, Y2, Z, Z2]:
            pl.semaphore_signal(
                barrier_sem, inc=1,
                device_id=(d ^ xor_const,),
                device_id_type=pl.DeviceIdType.MESH,
            )
        pl.semaphore_wait(barrier_sem, 5)

        x_v = x_ref[...]
        xp = jnp.concatenate(
            [jnp.zeros((B, TAPS - 1, C), jnp.float32), x_v], axis=1
        )
        w_v = w_ref[...]

        def compute_chunk(c):
            base = c * ROWS
            acc = xp[:, base:base + ROWS, :] * k_ref[0][None, None, :]
            for t in range(1, TAPS):
                acc = acc + xp[:, base + t:base + t + ROWS, :] * (
                    k_ref[t][None, None, :]
                )
            a = acc / (1.0 + jnp.exp(-acc))
            out_ref[:, base:base + ROWS, :] = jnp.dot(
                a.reshape(B * ROWS, C), w_v,
                preferred_element_type=jnp.float32,
            ).reshape(B, ROWS, C)

        seg = [None, None]
        rdmas = [[None] * N_OPS, [None] * N_OPS]

        def op_params(c, j):
            if j < 5:
                xor_const, shift = ORDERS[c][j]
                return xor_const, shift, HALVES[j]
            xor_const, shift = ORDERS[c][9 - j]
            return xor_const, shift, HALVES[9 - j]

        def start_op(c, j):
            xor_const, shift, h = op_params(c, j)
            sem = c * N_OPS + j
            if j < 5:
                bit = (d >> shift) & 1
                send_start = seg[c] + (1 - bit) * h
                src = out_ref.at[:, pl.ds(send_start, h), :]
                dst = comm_ref.at[c, :, pl.ds(COMM_OFF[j], h), :]
            else:
                src = out_ref.at[:, pl.ds(seg[c], h), :]
                dst = out_ref.at[:, pl.ds(seg[c], h), :]
            rdmas[c][j] = pltpu.make_async_remote_copy(
                src_ref=src, dst_ref=dst,
                send_sem=send_sems.at[sem], recv_sem=recv_sems.at[sem],
                device_id=(d ^ xor_const,),
                device_id_type=pl.DeviceIdType.MESH,
            )
            rdmas[c][j].start()

        def finish_op(c, j):
            xor_const, shift, h = op_params(c, j)
            rdmas[c][j].wait()
            bit = (d >> shift) & 1
            if j < 5:
                keep_start = seg[c] + bit * h
                out_ref[:, pl.ds(keep_start, h), :] = (
                    out_ref[:, pl.ds(keep_start, h), :]
                    + comm_ref[c, :, pl.ds(COMM_OFF[j], h), :]
                )
                seg[c] = keep_start
            else:
                seg[c] = seg[c] - bit * h

        for c in range(N_CHUNK):
            compute_chunk(c)
            seg[c] = jnp.int32(c * ROWS)
            start_op(c, 0)
        for j in range(N_OPS):
            for c in range(N_CHUNK):
                finish_op(c, j)
                if j + 1 < N_OPS:
                    start_op(c, j + 1)

    return pl.pallas_call(
        body,
        out_shape=jax.ShapeDtypeStruct((B, S, C), jnp.float32),
        in_specs=[
            pl.BlockSpec(memory_space=pltpu.VMEM),
            pl.BlockSpec(memory_space=pltpu.VMEM),
            pl.BlockSpec(memory_space=pltpu.VMEM),
        ],
        out_specs=pl.BlockSpec(memory_space=pltpu.VMEM),
        scratch_shapes=[
            pltpu.VMEM((N_CHUNK, B, COMM_ROWS, C), jnp.float32),
            pltpu.SemaphoreType.DMA((N_CHUNK * N_OPS,)),
            pltpu.SemaphoreType.DMA((N_CHUNK * N_OPS,)),
        ],
        compiler_params=pltpu.CompilerParams(collective_id=0),
    )(x, k, Wp)


# device time: 51460 ns/iter; 4.9027x vs baseline; 1.0223x over previous
import jax
import jax.numpy as jnp
from jax import lax
from jax.experimental import pallas as pl
from jax.experimental.pallas import tpu as pltpu

N_DEV = 32
B, S, C = 4, 512, 256
TAPS = 4

N_CHUNK = 4
ROWS = S // 2
LANEW = C // 2

X, DIAG, Y2, Z, Z2 = (1, 0), (2, 1), (4, 2), (8, 3), (16, 4)
ORDERS = [
    [X, Z, DIAG, Y2, Z2],
    [Z, X, Z2, DIAG, Y2],
    [DIAG, Y2, X, Z2, Z],
    [Y2, Z2, Z, X, DIAG],
]
HALVES = [ROWS >> (s + 1) for s in range(5)]
COMM_OFF = [sum(HALVES[:s]) for s in range(5)]
COMM_ROWS = sum(HALVES)
N_OPS = 10


def kernel(x, k, Wp):
    def body(x_ref, k_ref, w_ref, out_ref, comm_ref, send_sems, recv_sems):
        d = lax.axis_index("i")

        barrier_sem = pltpu.get_barrier_semaphore()
        for xor_const, _ in [X, DIAG, Y2, Z, Z2]:
            pl.semaphore_signal(
                barrier_sem, inc=1,
                device_id=(d ^ xor_const,),
                device_id_type=pl.DeviceIdType.MESH,
            )
        pl.semaphore_wait(barrier_sem, 5)

        x_v = x_ref[...]
        xp = jnp.concatenate(
            [jnp.zeros((B, TAPS - 1, C), jnp.float32), x_v], axis=1
        )
        w_v = w_ref[...]

        def compute_rows(base):
            acc = xp[:, base:base + ROWS, :] * k_ref[0][None, None, :]
            for t in range(1, TAPS):
                acc = acc + xp[:, base + t:base + t + ROWS, :] * (
                    k_ref[t][None, None, :]
                )
            a = acc / (1.0 + jnp.exp(-acc))
            out_ref[:, base:base + ROWS, :] = jnp.dot(
                a.reshape(B * ROWS, C), w_v,
                preferred_element_type=jnp.float32,
            ).reshape(B, ROWS, C)

        seg = [None] * N_CHUNK
        rdmas = [[None] * N_OPS for _ in range(N_CHUNK)]

        def op_params(c, j):
            if j < 5:
                xor_const, shift = ORDERS[c][j]
                return xor_const, shift, HALVES[j]
            xor_const, shift = ORDERS[c][9 - j]
            return xor_const, shift, HALVES[9 - j]

        def start_op(c, j):
            xor_const, shift, h = op_params(c, j)
            lb = (c // 2) * LANEW
            sem = c * N_OPS + j
            if j < 5:
                bit = (d >> shift) & 1
                send_start = seg[c] + (1 - bit) * h
                src = out_ref.at[:, pl.ds(send_start, h), pl.ds(lb, LANEW)]
                dst = comm_ref.at[c, :, pl.ds(COMM_OFF[j], h), :]
            else:
                src = out_ref.at[:, pl.ds(seg[c], h), pl.ds(lb, LANEW)]
                dst = src
            rdmas[c][j] = pltpu.make_async_remote_copy(
                src_ref=src, dst_ref=dst,
                send_sem=send_sems.at[sem], recv_sem=recv_sems.at[sem],
                device_id=(d ^ xor_const,),
                device_id_type=pl.DeviceIdType.MESH,
            )
            rdmas[c][j].start()

        def finish_op(c, j):
            xor_const, shift, h = op_params(c, j)
            lb = (c // 2) * LANEW
            rdmas[c][j].wait()
            bit = (d >> shift) & 1
            if j < 5:
                keep_start = seg[c] + bit * h
                out_ref[:, pl.ds(keep_start, h), pl.ds(lb, LANEW)] = (
                    out_ref[:, pl.ds(keep_start, h), pl.ds(lb, LANEW)]
                    + comm_ref[c, :, pl.ds(COMM_OFF[j], h), :]
                )
                seg[c] = keep_start
            else:
                seg[c] = seg[c] - bit * h

        compute_rows(0)
        for c in (0, 2):
            seg[c] = jnp.int32(0)
            start_op(c, 0)
        compute_rows(ROWS)
        for c in (1, 3):
            seg[c] = jnp.int32(ROWS)
            start_op(c, 0)
        for j in range(N_OPS):
            for c in range(N_CHUNK):
                finish_op(c, j)
                if j + 1 < N_OPS:
                    start_op(c, j + 1)

    return pl.pallas_call(
        body,
        out_shape=jax.ShapeDtypeStruct((B, S, C), jnp.float32),
        in_specs=[
            pl.BlockSpec(memory_space=pltpu.VMEM),
            pl.BlockSpec(memory_space=pltpu.VMEM),
            pl.BlockSpec(memory_space=pltpu.VMEM),
        ],
        out_specs=pl.BlockSpec(memory_space=pltpu.VMEM),
        scratch_shapes=[
            pltpu.VMEM((N_CHUNK, B, COMM_ROWS, LANEW), jnp.float32),
            pltpu.SemaphoreType.DMA((N_CHUNK * N_OPS,)),
            pltpu.SemaphoreType.DMA((N_CHUNK * N_OPS,)),
        ],
        compiler_params=pltpu.CompilerParams(collective_id=0),
    )(x, k, Wp)


# device time: 32481 ns/iter; 7.7674x vs baseline; 1.5843x over previous
import jax
import jax.numpy as jnp
from jax import lax
from jax.experimental import pallas as pl
from jax.experimental.pallas import tpu as pltpu

N_DEV = 32
B, S, C = 4, 512, 256
TAPS = 4

N_CHUNK = 8
N_Q = 4
ROWS = S // N_Q
LANEW = C // 2
BSEG = ROWS // 4

X, DIAG, Y2, Z, Z2 = (1, 0), (2, 1), (4, 2), (8, 3), (16, 4)
HALVE_ORDERS = [
    [X, Z], [Z, X], [X, Z], [Z, X],
    [Z, X], [X, Z], [Z, X], [X, Z],
]
BF_ORDERS = [
    [DIAG, Y2, Z2],
    [Z2, DIAG, Y2],
    [Y2, Z2, DIAG],
    [DIAG, Z2, Y2],
    [Y2, DIAG, Z2],
    [Z2, Y2, DIAG],
    [DIAG, Y2, Z2],
    [Y2, Z2, DIAG],
]
HALVES = [64, 32]
N_OPS = 7
OP_ROWS = [64, 32, 32, 32, 32, 32, 64]
OP_OFF = [0, 64, 96, 128, 160, 192, 224]
COMM_ROWS = 288


def kernel(x, k, Wp):
    def body(x_ref, k_ref, w_ref, out_ref, stage_ref, comm_ref, send_sems,
             recv_sems):
        d = lax.axis_index("i")

        barrier_sem = pltpu.get_barrier_semaphore()
        for xor_const in (1, 2, 4, 8, 16):
            pl.semaphore_signal(
                barrier_sem, inc=1,
                device_id=(d ^ xor_const,),
                device_id_type=pl.DeviceIdType.MESH,
            )

        x_v = x_ref[...]
        xp = jnp.concatenate(
            [jnp.zeros((B, TAPS - 1, C), jnp.float32), x_v], axis=1
        )
        w_bf = w_ref[...].astype(jnp.bfloat16)

        def compute_rows(base):
            acc = xp[:, base:base + ROWS, :] * k_ref[0][None, None, :]
            for t in range(1, TAPS):
                acc = acc + xp[:, base + t:base + t + ROWS, :] * (
                    k_ref[t][None, None, :]
                )
            a = acc / (1.0 + jnp.exp(-acc))
            out_ref[:, base:base + ROWS, :] = jnp.dot(
                a.reshape(B * ROWS, C).astype(jnp.bfloat16), w_bf,
                preferred_element_type=jnp.float32,
            ).reshape(B, ROWS, C)

        seg = [None] * N_CHUNK
        rdmas = [[None] * N_OPS for _ in range(N_CHUNK)]

        def op_params(c, j):
            if j < 2:
                xor_const, shift = HALVE_ORDERS[c][j]
                return xor_const, shift, HALVES[j], "rs"
            if j < 5:
                xor_const, shift = BF_ORDERS[c][j - 2]
                return xor_const, shift, BSEG, "bf"
            xor_const, shift = HALVE_ORDERS[c][6 - j]
            return xor_const, shift, HALVES[6 - j], "ag"

        def start_op(c, j):
            xor_const, shift, h, kind = op_params(c, j)
            lb = (c // N_Q) * LANEW
            if kind == "rs":
                bit = (d >> shift) & 1
                send_start = seg[c] + (1 - bit) * h
            else:
                send_start = seg[c]
            stage_ref[c, :, pl.ds(OP_OFF[j], h), :] = out_ref[
                :, pl.ds(send_start, h), pl.ds(lb, LANEW)
            ].astype(jnp.bfloat16)
            sem = c * N_OPS + j
            rdmas[c][j] = pltpu.make_async_remote_copy(
                src_ref=stage_ref.at[c, :, pl.ds(OP_OFF[j], h), :],
                dst_ref=comm_ref.at[c, :, pl.ds(OP_OFF[j], h), :],
                send_sem=send_sems.at[sem], recv_sem=recv_sems.at[sem],
                device_id=(d ^ xor_const,),
                device_id_type=pl.DeviceIdType.MESH,
            )
            rdmas[c][j].start()

        def finish_op(c, j):
            xor_const, shift, h, kind = op_params(c, j)
            lb = (c // N_Q) * LANEW
            rdmas[c][j].wait_recv()
            bit = (d >> shift) & 1
            recv = comm_ref[c, :, pl.ds(OP_OFF[j], h), :].astype(jnp.float32)
            if kind == "rs":
                keep_start = seg[c] + bit * h
                out_ref[:, pl.ds(keep_start, h), pl.ds(lb, LANEW)] = (
                    out_ref[:, pl.ds(keep_start, h), pl.ds(lb, LANEW)] + recv
                )
                seg[c] = keep_start
            elif kind == "bf":
                out_ref[:, pl.ds(seg[c], h), pl.ds(lb, LANEW)] = (
                    out_ref[:, pl.ds(seg[c], h), pl.ds(lb, LANEW)] + recv
                )
            else:
                partner_start = seg[c] - bit * h + (1 - bit) * h
                out_ref[:, pl.ds(partner_start, h), pl.ds(lb, LANEW)] = recv
                seg[c] = seg[c] - bit * h

        for q in range(N_Q):
            compute_rows(q * ROWS)
            if q == 0:
                pl.semaphore_wait(barrier_sem, 5)
            for c in (q, q + N_Q):
                seg[c] = jnp.int32(q * ROWS)
                start_op(c, 0)
        for j in range(N_OPS):
            for c in range(N_CHUNK):
                finish_op(c, j)
                if j + 1 < N_OPS:
                    start_op(c, j + 1)
        for c in range(N_CHUNK):
            for j in range(N_OPS):
                rdmas[c][j].wait_send()

    return pl.pallas_call(
        body,
        out_shape=jax.ShapeDtypeStruct((B, S, C), jnp.float32),
        in_specs=[
            pl.BlockSpec(memory_space=pltpu.VMEM),
            pl.BlockSpec(memory_space=pltpu.VMEM),
            pl.BlockSpec(memory_space=pltpu.VMEM),
        ],
        out_specs=pl.BlockSpec(memory_space=pltpu.VMEM),
        scratch_shapes=[
            pltpu.VMEM((N_CHUNK, B, COMM_ROWS, LANEW), jnp.bfloat16),
            pltpu.VMEM((N_CHUNK, B, COMM_ROWS, LANEW), jnp.bfloat16),
            pltpu.SemaphoreType.DMA((N_CHUNK * N_OPS,)),
            pltpu.SemaphoreType.DMA((N_CHUNK * N_OPS,)),
        ],
        compiler_params=pltpu.CompilerParams(collective_id=0),
    )(x, k, Wp)
